# Initial kernel scaffold; baseline (speedup 1.0000x reference)
import functools

import jax
import jax.numpy as jnp
from jax import lax
from jax.experimental import pallas as pl
from jax.experimental.pallas import tpu as pltpu
from jax.experimental.pallas import tpu_sc as plsc

N = 10000
E = 320000
R = 8
D = 128
NB = 2
BI = D // NB

NC = 2
NS = 16
NW = NC * NS

CH = 128
EP = 323584
PADE = EP - E
CPT_CNT = EP // (NS * CH)
CPT_MSG = EP // (NW * CH)
EHALF = EP // NC
ETILE = EP // NW

RNP = 80128
RN = R * N
CSL = RNP // NS
NPT = N // NS

_mesh = plsc.VectorSubcoreMesh(core_axis_name="c", subcore_axis_name="s")


def _fill16(ref, n16, value):
  vec = jnp.full((16,), value, jnp.float32)

  def body(i, _):
    ref[pl.ds(i * 16, 16)] = vec
    return 0

  lax.fori_loop(0, n16, body, 0)


@functools.partial(
    pl.kernel,
    out_type=(
        jax.ShapeDtypeStruct((EP,), jnp.float32),
        jax.ShapeDtypeStruct((EP,), jnp.int32),
    ),
    mesh=_mesh,
    scratch_types=dict(
        counts_sh=pltpu.VMEM_SHARED((RNP,), jnp.float32),
        inv_vm=pltpu.VMEM((RNP,), jnp.float32),
        et_b=pltpu.VMEM((CH,), jnp.int32),
        dst_b=pltpu.VMEM((CH,), jnp.int32),
        src_b=pltpu.VMEM((CH,), jnp.int32),
        comb_b=pltpu.VMEM((CH,), jnp.int32),
        csrc_b=pltpu.VMEM((CH,), jnp.int32),
        norm_b=pltpu.VMEM((CH,), jnp.float32),
        ones_b=pltpu.VMEM((CH,), jnp.float32),
        zero_b=pltpu.VMEM((CSL,), jnp.float32),
    ),
)
def _sc_prep(src_hbm, dst_hbm, et_hbm, norm_hbm, csrc_hbm,
             counts_sh, inv_vm, et_b, dst_b, src_b, comb_b, csrc_b, norm_b,
             ones_b, zero_b):
  c = lax.axis_index("c")
  s = lax.axis_index("s")

  _fill16(ones_b, CH // 16, 1.0)
  _fill16(zero_b, CSL // 16, 0.0)

  pltpu.sync_copy(zero_b, counts_sh.at[pl.ds(s * CSL, CSL)])
  plsc.subcore_barrier()

  def count_chunk(j, _):
    off = s * (CPT_CNT * CH) + j * CH
    pltpu.sync_copy(et_hbm.at[pl.ds(off, CH)], et_b)
    pltpu.sync_copy(dst_hbm.at[pl.ds(off, CH)], dst_b)
    for q in range(CH // 16):
      sl = pl.ds(q * 16, 16)
      comb_b[sl] = et_b[sl] * N + dst_b[sl]
    pltpu.sync_copy(ones_b, counts_sh.at[comb_b], add=True)
    return 0

  lax.fori_loop(0, CPT_CNT, count_chunk, 0)
  plsc.subcore_barrier()

  o = s * CSL
  pltpu.sync_copy(counts_sh.at[pl.ds(o, CSL)], inv_vm.at[pl.ds(o, CSL)])

  def inv_body(q, _):
    sl = pl.ds(o + q * 16, 16)
    cnt = inv_vm[sl]
    val = 1.0 / jnp.maximum(cnt, 1.0)
    g0 = o + q * 16
    inv_vm[sl] = jnp.where(g0 < RN, val, jnp.zeros_like(val))
    return 0

  lax.fori_loop(0, CSL // 16, inv_body, 0)
  pltpu.sync_copy(inv_vm.at[pl.ds(o, CSL)], counts_sh.at[pl.ds(o, CSL)])
  plsc.subcore_barrier()

  pltpu.sync_copy(counts_sh, inv_vm)

  def norm_chunk(j, _):
    off = c * EHALF + s * ETILE + j * CH
    pltpu.sync_copy(et_hbm.at[pl.ds(off, CH)], et_b)
    pltpu.sync_copy(dst_hbm.at[pl.ds(off, CH)], dst_b)
    pltpu.sync_copy(src_hbm.at[pl.ds(off, CH)], src_b)
    for q in range(CH // 16):
      sl = pl.ds(q * 16, 16)
      et16 = et_b[sl]
      combd = et16 * N + dst_b[sl]
      csrc_b[sl] = jnp.minimum(et16, R - 1) * N + src_b[sl]
      norm_b[sl] = plsc.load_gather(inv_vm, [combd])
    pltpu.sync_copy(csrc_b, csrc_hbm.at[pl.ds(off, CH)])
    pltpu.sync_copy(norm_b, norm_hbm.at[pl.ds(off, CH)])
    return 0

  lax.fori_loop(0, CPT_MSG, norm_chunk, 0)


@functools.partial(
    pl.kernel,
    out_type=jax.ShapeDtypeStruct((NC * N, D), jnp.float32),
    mesh=_mesh,
    scratch_types=dict(
        acc=pltpu.VMEM_SHARED((N, D), jnp.float32),
        rows=pltpu.VMEM((CH, D), jnp.float32),
        idx_b=pltpu.VMEM((CH,), jnp.int32),
        dst_b=pltpu.VMEM((CH,), jnp.int32),
        norm_b=pltpu.VMEM((CH,), jnp.float32),
        sem=pltpu.SemaphoreType.DMA,
    ),
)
def _sc_message(xs_hbm, csrc_hbm, dst_hbm, norm_hbm, out_hbm,
                acc, rows, idx_b, dst_b, norm_b, sem):
  c = lax.axis_index("c")
  s = lax.axis_index("s")

  def zrow(i, _):
    for q in range(D // 16):
      rows[i, pl.ds(q * 16, 16)] = jnp.zeros((16,), jnp.float32)
    return 0

  lax.fori_loop(0, CH, zrow, 0)
  for piece in range(5):
    pltpu.sync_copy(rows.at[pl.ds(0, 125)],
                    acc.at[pl.ds(s * NPT + piece * 125, 125)])
  plsc.subcore_barrier()

  def msg_chunk(j, _):
    off = c * EHALF + s * ETILE + j * CH
    pltpu.sync_copy(csrc_hbm.at[pl.ds(off, CH)], idx_b)
    pltpu.sync_copy(dst_hbm.at[pl.ds(off, CH)], dst_b)
    pltpu.sync_copy(norm_hbm.at[pl.ds(off, CH)], norm_b)
    pltpu.async_copy(xs_hbm.at[idx_b], rows, sem).wait()

    def scale(k, _):
      nk = norm_b[k]
      for q in range(D // 16):
        sl = pl.ds(q * 16, 16)
        rows[k, sl] = rows[k, sl] * nk
      return 0

    lax.fori_loop(0, CH, scale, 0)
    pltpu.sync_copy(rows, acc.at[dst_b], add=True)
    return 0

  lax.fori_loop(0, CPT_MSG, msg_chunk, 0)
  plsc.subcore_barrier()

  pltpu.sync_copy(acc.at[pl.ds(s * NPT, NPT)],
                  out_hbm.at[pl.ds(c * N + s * NPT, NPT)])


_BN = 1000


def _tc_transform_body(h_ref, w_ref, root_ref, b_ref, xs_ref, base_ref):
  h = h_ref[...]
  base_ref[...] = (
      jnp.dot(h, root_ref[...], preferred_element_type=jnp.float32)
      + b_ref[...]
  )
  for r in range(R):
    for blk in range(NB):
      xs_ref[r, :, blk * BI:(blk + 1) * BI] = jnp.dot(
          h[:, blk * BI:(blk + 1) * BI], w_ref[r, blk],
          preferred_element_type=jnp.float32)


def _tc_transform(h, w, root, b):
  xs, base = pl.pallas_call(
      _tc_transform_body,
      grid=(N // _BN,),
      in_specs=[
          pl.BlockSpec((_BN, D), lambda i: (i, 0)),
          pl.BlockSpec((R, NB, BI, BI), lambda i: (0, 0, 0, 0)),
          pl.BlockSpec((D, D), lambda i: (0, 0)),
          pl.BlockSpec((1, D), lambda i: (0, 0)),
      ],
      out_specs=[
          pl.BlockSpec((R, _BN, D), lambda i: (0, i, 0)),
          pl.BlockSpec((_BN, D), lambda i: (i, 0)),
      ],
      out_shape=[
          jax.ShapeDtypeStruct((R, N, D), jnp.float32),
          jax.ShapeDtypeStruct((N, D), jnp.float32),
      ],
  )(h, w, root, b.reshape(1, D))
  return xs.reshape(R * N, D), base


def _tc_combine(p0, p1, base, relu):
  def body(p0_ref, p1_ref, base_ref, out_ref):
    t = p0_ref[...] + p1_ref[...] + base_ref[...]
    out_ref[...] = jnp.maximum(t, 0.0) if relu else t

  return pl.pallas_call(
      body,
      grid=(N // _BN,),
      in_specs=[pl.BlockSpec((_BN, D), lambda i: (i, 0))] * 3,
      out_specs=pl.BlockSpec((_BN, D), lambda i: (i, 0)),
      out_shape=jax.ShapeDtypeStruct((N, D), jnp.float32),
  )(p0, p1, base)


def kernel(x, edge_index, edge_type, w1, root1, b1, w2, root2, b2):
  src = edge_index[0].astype(jnp.int32)
  dst = edge_index[1].astype(jnp.int32)
  et = edge_type.astype(jnp.int32)
  srcp = jnp.concatenate([src, jnp.zeros((PADE,), jnp.int32)])
  dstp = jnp.concatenate([dst, jnp.zeros((PADE,), jnp.int32)])
  etp = jnp.concatenate([et, jnp.full((PADE,), R, jnp.int32)])

  norm, csrc = _sc_prep(srcp, dstp, etp)

  xs1, base1 = _tc_transform(x, w1, root1, b1)
  parts1 = _sc_message(xs1, csrc, dstp, norm)
  h = _tc_combine(parts1[:N], parts1[N:], base1, relu=True)

  xs2, base2 = _tc_transform(h, w2, root2, b2)
  parts2 = _sc_message(xs2, csrc, dstp, norm)
  return _tc_combine(parts2[:N], parts2[N:], base2, relu=False)


# speedup vs baseline: 11.4198x; 11.4198x over previous
import functools

import jax
import jax.numpy as jnp
from jax import lax
from jax.experimental import pallas as pl
from jax.experimental.pallas import tpu as pltpu
from jax.experimental.pallas import tpu_sc as plsc

N = 10000
E = 320000
R = 8
D = 128
NB = 2
BI = D // NB

NC = 2
NS = 16
NW = NC * NS

CH = 128
EP = 323584
PADE = EP - E
CPT_CNT = EP // (NS * CH)
CPT_MSG = EP // (NW * CH)
EHALF = EP // NC
ETILE = EP // NW

RNP = 80128
RN = R * N
CSL = RNP // NS
NPAD = 10240
NPT = NPAD // NS

_mesh = plsc.VectorSubcoreMesh(core_axis_name="c", subcore_axis_name="s")


def _fill16(ref, n16, value):
  vec = jnp.full((16,), value, jnp.float32)

  def body(i, _):
    ref[pl.ds(i * 16, 16)] = vec
    return 0

  lax.fori_loop(0, n16, body, 0)


@functools.partial(
    pl.kernel,
    out_type=(
        jax.ShapeDtypeStruct((EP,), jnp.float32),
        jax.ShapeDtypeStruct((EP,), jnp.int32),
    ),
    mesh=_mesh,
    compiler_params=pltpu.CompilerParams(needs_layout_passes=False),
    scratch_types=dict(
        counts_sh=pltpu.VMEM_SHARED((RNP,), jnp.float32),
        inv_vm=pltpu.VMEM((RNP,), jnp.float32),
        et_b=pltpu.VMEM((CH,), jnp.int32),
        dst_b=pltpu.VMEM((CH,), jnp.int32),
        src_b=pltpu.VMEM((CH,), jnp.int32),
        comb_b=pltpu.VMEM((CH,), jnp.int32),
        csrc_b=pltpu.VMEM((CH,), jnp.int32),
        norm_b=pltpu.VMEM((CH,), jnp.float32),
        ones_b=pltpu.VMEM((CH,), jnp.float32),
        zero_b=pltpu.VMEM((CSL,), jnp.float32),
    ),
)
def _sc_prep(src_hbm, dst_hbm, et_hbm, norm_hbm, csrc_hbm,
             counts_sh, inv_vm, et_b, dst_b, src_b, comb_b, csrc_b, norm_b,
             ones_b, zero_b):
  c = lax.axis_index("c")
  s = lax.axis_index("s")

  _fill16(ones_b, CH // 16, 1.0)
  _fill16(zero_b, CSL // 16, 0.0)

  pltpu.sync_copy(zero_b, counts_sh.at[pl.ds(s * CSL, CSL)])
  plsc.subcore_barrier()

  def count_chunk(j, _):
    off = s * (CPT_CNT * CH) + j * CH
    pltpu.sync_copy(et_hbm.at[pl.ds(off, CH)], et_b)
    pltpu.sync_copy(dst_hbm.at[pl.ds(off, CH)], dst_b)
    for q in range(CH // 16):
      sl = pl.ds(q * 16, 16)
      comb_b[sl] = et_b[sl] * N + dst_b[sl]
    pltpu.sync_copy(ones_b, counts_sh.at[comb_b], add=True)
    return 0

  lax.fori_loop(0, CPT_CNT, count_chunk, 0)
  plsc.subcore_barrier()

  o = s * CSL
  pltpu.sync_copy(counts_sh.at[pl.ds(o, CSL)], inv_vm.at[pl.ds(o, CSL)])

  def inv_body(q, _):
    sl = pl.ds(o + q * 16, 16)
    cnt = inv_vm[sl]
    val = 1.0 / jnp.maximum(cnt, 1.0)
    g0 = o + q * 16
    inv_vm[sl] = jnp.where(g0 < RN, val, jnp.zeros_like(val))
    return 0

  lax.fori_loop(0, CSL // 16, inv_body, 0)
  pltpu.sync_copy(inv_vm.at[pl.ds(o, CSL)], counts_sh.at[pl.ds(o, CSL)])
  plsc.subcore_barrier()

  pltpu.sync_copy(counts_sh, inv_vm)

  def norm_chunk(j, _):
    off = c * EHALF + s * ETILE + j * CH
    pltpu.sync_copy(et_hbm.at[pl.ds(off, CH)], et_b)
    pltpu.sync_copy(dst_hbm.at[pl.ds(off, CH)], dst_b)
    pltpu.sync_copy(src_hbm.at[pl.ds(off, CH)], src_b)
    for q in range(CH // 16):
      sl = pl.ds(q * 16, 16)
      et16 = et_b[sl]
      combd = et16 * N + dst_b[sl]
      csrc_b[sl] = jnp.minimum(et16, R - 1) * N + src_b[sl]
      norm_b[sl] = plsc.load_gather(inv_vm, [combd])
    pltpu.sync_copy(csrc_b, csrc_hbm.at[pl.ds(off, CH)])
    pltpu.sync_copy(norm_b, norm_hbm.at[pl.ds(off, CH)])
    return 0

  lax.fori_loop(0, CPT_MSG, norm_chunk, 0)


@functools.partial(
    pl.kernel,
    out_type=jax.ShapeDtypeStruct((NC * NPAD, D), jnp.float32),
    mesh=_mesh,
    compiler_params=pltpu.CompilerParams(needs_layout_passes=False),
    scratch_types=dict(
        acc=pltpu.VMEM_SHARED((NPAD, D), jnp.float32),
        rows=pltpu.VMEM((CH, D), jnp.float32),
        idx_b=pltpu.VMEM((CH,), jnp.int32),
        dst_b=pltpu.VMEM((CH,), jnp.int32),
        norm_b=pltpu.VMEM((CH,), jnp.float32),
        sem=pltpu.SemaphoreType.DMA,
    ),
)
def _sc_message(xs_hbm, csrc_hbm, dst_hbm, norm_hbm, out_hbm,
                acc, rows, idx_b, dst_b, norm_b, sem):
  c = lax.axis_index("c")
  s = lax.axis_index("s")

  def zrow(i, _):
    for q in range(D // 16):
      rows[i, pl.ds(q * 16, 16)] = jnp.zeros((16,), jnp.float32)
    return 0

  lax.fori_loop(0, CH, zrow, 0)
  for piece in range(NPT // CH):
    pltpu.sync_copy(rows, acc.at[pl.ds(s * NPT + piece * CH, CH)])
  plsc.subcore_barrier()

  def msg_chunk(j, _):
    off = c * EHALF + s * ETILE + j * CH
    pltpu.sync_copy(csrc_hbm.at[pl.ds(off, CH)], idx_b)
    pltpu.sync_copy(dst_hbm.at[pl.ds(off, CH)], dst_b)
    pltpu.sync_copy(norm_hbm.at[pl.ds(off, CH)], norm_b)
    pltpu.async_copy(xs_hbm.at[idx_b], rows, sem).wait()

    def scale(t, _):
      nv = norm_b[pl.ds(t * 16, 16)]
      for k16 in range(16):
        nk = nv[k16]
        for q in range(D // 16):
          sl = pl.ds(q * 16, 16)
          rows[t * 16 + k16, sl] = rows[t * 16 + k16, sl] * nk
      return 0

    lax.fori_loop(0, CH // 16, scale, 0)
    pltpu.sync_copy(rows, acc.at[dst_b], add=True)
    return 0

  lax.fori_loop(0, CPT_MSG, msg_chunk, 0)
  plsc.subcore_barrier()

  pltpu.sync_copy(acc.at[pl.ds(s * NPT, NPT)],
                  out_hbm.at[pl.ds(c * NPAD + s * NPT, NPT)])


_BN = 1000


def _tc_transform_body(h_ref, w_ref, root_ref, b_ref, xs_ref, base_ref):
  h = h_ref[...]
  base_ref[...] = (
      jnp.dot(h, root_ref[...], preferred_element_type=jnp.float32)
      + b_ref[...]
  )
  for r in range(R):
    for blk in range(NB):
      xs_ref[r, :, blk * BI:(blk + 1) * BI] = jnp.dot(
          h[:, blk * BI:(blk + 1) * BI], w_ref[r, blk],
          preferred_element_type=jnp.float32)


def _tc_transform(h, w, root, b):
  xs, base = pl.pallas_call(
      _tc_transform_body,
      grid=(N // _BN,),
      in_specs=[
          pl.BlockSpec((_BN, D), lambda i: (i, 0)),
          pl.BlockSpec((R, NB, BI, BI), lambda i: (0, 0, 0, 0)),
          pl.BlockSpec((D, D), lambda i: (0, 0)),
          pl.BlockSpec((1, D), lambda i: (0, 0)),
      ],
      out_specs=[
          pl.BlockSpec((R, _BN, D), lambda i: (0, i, 0)),
          pl.BlockSpec((_BN, D), lambda i: (i, 0)),
      ],
      out_shape=[
          jax.ShapeDtypeStruct((R, N, D), jnp.float32),
          jax.ShapeDtypeStruct((N, D), jnp.float32),
      ],
  )(h, w, root, b.reshape(1, D))
  return xs.reshape(R * N, D), base


def _tc_combine(p0, p1, base, relu):
  def body(p0_ref, p1_ref, base_ref, out_ref):
    t = p0_ref[...] + p1_ref[...] + base_ref[...]
    out_ref[...] = jnp.maximum(t, 0.0) if relu else t

  return pl.pallas_call(
      body,
      grid=(N // _BN,),
      in_specs=[pl.BlockSpec((_BN, D), lambda i: (i, 0))] * 3,
      out_specs=pl.BlockSpec((_BN, D), lambda i: (i, 0)),
      out_shape=jax.ShapeDtypeStruct((N, D), jnp.float32),
  )(p0, p1, base)


def kernel(x, edge_index, edge_type, w1, root1, b1, w2, root2, b2):
  src = edge_index[0].astype(jnp.int32)
  dst = edge_index[1].astype(jnp.int32)
  et = edge_type.astype(jnp.int32)
  srcp = jnp.concatenate([src, jnp.zeros((PADE,), jnp.int32)])
  dstp = jnp.concatenate([dst, jnp.zeros((PADE,), jnp.int32)])
  etp = jnp.concatenate([et, jnp.full((PADE,), R, jnp.int32)])

  norm, csrc = _sc_prep(srcp, dstp, etp)

  xs1, base1 = _tc_transform(x, w1, root1, b1)
  parts1 = _sc_message(xs1, csrc, dstp, norm)
  h = _tc_combine(parts1[:N], parts1[NPAD:NPAD + N], base1, relu=True)

  xs2, base2 = _tc_transform(h, w2, root2, b2)
  parts2 = _sc_message(xs2, csrc, dstp, norm)
  return _tc_combine(parts2[:N], parts2[NPAD:NPAD + N], base2, relu=False)


# speedup vs baseline: 12.8498x; 1.1252x over previous
import functools

import jax
import jax.numpy as jnp
from jax import lax
from jax.experimental import pallas as pl
from jax.experimental.pallas import tpu as pltpu
from jax.experimental.pallas import tpu_sc as plsc

N = 10000
E = 320000
R = 8
D = 128
NB = 2
BI = D // NB

NC = 2
NS = 16
NW = NC * NS

CH = 128
EP = 327680
PADE = EP - E
EROWS = EP // CH
CPT_MSG = EP // (NW * CH)
EHALF = EP // NC
ETILE = EP // NW
CROWS = ETILE // CH

SUB = 16
SUBE = SUB * CH

RNP = 80128
RN = R * N
CSL = RNP // NS
NPAD = 10240
NPT = NPAD // NS

_mesh = plsc.VectorSubcoreMesh(core_axis_name="c", subcore_axis_name="s")


def _fill16(ref, n16, value):
  vec = jnp.full((16,), value, jnp.float32)

  def body(i, _):
    ref[pl.ds(i * 16, 16)] = vec
    return 0

  lax.fori_loop(0, n16, body, 0)


@functools.partial(
    pl.kernel,
    out_type=(
        jax.ShapeDtypeStruct((EP,), jnp.int32),
        jax.ShapeDtypeStruct((EP,), jnp.int32),
    ),
    mesh=_mesh,
    compiler_params=pltpu.CompilerParams(needs_layout_passes=False),
    scratch_types=dict(
        counts_sh=pltpu.VMEM_SHARED((RNP,), jnp.float32),
        inv_vm=pltpu.VMEM((RNP,), jnp.float32),
        eta=pltpu.VMEM((SUBE,), jnp.int32),
        dsta=pltpu.VMEM((SUBE,), jnp.int32),
        srca=pltpu.VMEM((SUBE,), jnp.int32),
        comb2=pltpu.VMEM((SUB, CH), jnp.int32),
        ones_b=pltpu.VMEM((CH,), jnp.float32),
        zero_b=pltpu.VMEM((CSL // 2,), jnp.float32),
        sem=pltpu.SemaphoreType.DMA,
    ),
)
def _sc_prep(src_hbm, dst_hbm, et_hbm, norm_hbm, csrc_hbm,
             counts_sh, inv_vm, eta, dsta, srca, comb2, ones_b, zero_b, sem):
  c = lax.axis_index("c")
  s = lax.axis_index("s")

  _fill16(ones_b, CH // 16, 1.0)
  _fill16(zero_b, CSL // 32, 0.0)

  pltpu.sync_copy(zero_b, counts_sh.at[pl.ds(s * CSL, CSL // 2)])
  pltpu.sync_copy(zero_b, counts_sh.at[pl.ds(s * CSL + CSL // 2, CSL // 2)])
  plsc.subcore_barrier()

  for sb in range(2 * ETILE // SUBE):
    off0 = s * (2 * ETILE) + sb * SUBE
    pltpu.sync_copy(et_hbm.at[pl.ds(off0, SUBE)], eta)
    pltpu.sync_copy(dst_hbm.at[pl.ds(off0, SUBE)], dsta)

    def comb_row(j, _):
      for q in range(CH // 16):
        sl = pl.ds(j * CH + q * 16, 16)
        comb2[j, pl.ds(q * 16, 16)] = eta[sl] * N + dsta[sl]
      return 0

    lax.fori_loop(0, SUB, comb_row, 0)

    descs = []
    for u in range(SUB):
      descs.append(
          pltpu.async_copy(ones_b, counts_sh.at[comb2.at[u]], sem, add=True))
    for d in descs:
      d.wait()
  plsc.subcore_barrier()

  o = s * CSL
  pltpu.sync_copy(counts_sh.at[pl.ds(o, CSL)], inv_vm.at[pl.ds(o, CSL)])

  def inv_body(q, _):
    sl = pl.ds(o + q * 16, 16)
    cnt = inv_vm[sl]
    val = 1.0 / jnp.maximum(cnt, 1.0)
    g0 = o + q * 16
    inv_vm[sl] = jnp.where(g0 < RN, val, jnp.zeros_like(val))
    return 0

  lax.fori_loop(0, CSL // 16, inv_body, 0)
  pltpu.sync_copy(inv_vm.at[pl.ds(o, CSL)], counts_sh.at[pl.ds(o, CSL)])
  plsc.subcore_barrier()

  pltpu.sync_copy(counts_sh, inv_vm)

  for sb in range(ETILE // SUBE):
    e0 = c * EHALF + s * ETILE + sb * SUBE
    pltpu.sync_copy(et_hbm.at[pl.ds(e0, SUBE)], eta)
    pltpu.sync_copy(dst_hbm.at[pl.ds(e0, SUBE)], dsta)
    pltpu.sync_copy(src_hbm.at[pl.ds(e0, SUBE)], srca)

    def norm_body(t, _):
      sl = pl.ds(t * 16, 16)
      et16 = eta[sl]
      combd = et16 * N + dsta[sl]
      csrc16 = jnp.minimum(et16, R - 1) * N + srca[sl]
      nv = plsc.load_gather(inv_vm, [combd])
      srca[sl] = csrc16
      eta[sl] = plsc.bitcast(nv, jnp.int32)
      return 0

    lax.fori_loop(0, SUBE // 16, norm_body, 0)
    pltpu.sync_copy(srca, csrc_hbm.at[pl.ds(e0, SUBE)])
    pltpu.sync_copy(eta, norm_hbm.at[pl.ds(e0, SUBE)])


@functools.partial(
    pl.kernel,
    out_type=jax.ShapeDtypeStruct((NC * NPAD, D), jnp.float32),
    mesh=_mesh,
    compiler_params=pltpu.CompilerParams(needs_layout_passes=False),
    scratch_types=dict(
        acc=pltpu.VMEM_SHARED((NPAD, D), jnp.float32),
        csrc2=pltpu.VMEM((SUB, CH), jnp.int32),
        dst2=pltpu.VMEM((SUB, CH), jnp.int32),
        norm2=pltpu.VMEM((SUB, CH), jnp.float32),
        rows0=pltpu.VMEM((CH, D), jnp.float32),
        rows1=pltpu.VMEM((CH, D), jnp.float32),
        gsem0=pltpu.SemaphoreType.DMA,
        gsem1=pltpu.SemaphoreType.DMA,
        ssem0=pltpu.SemaphoreType.DMA,
        ssem1=pltpu.SemaphoreType.DMA,
    ),
)
def _sc_message(xs_hbm, csrc_hbm, dst_hbm, norm_hbm, out_hbm,
                acc, csrc2, dst2, norm2, rows0, rows1,
                gsem0, gsem1, ssem0, ssem1):
  c = lax.axis_index("c")
  s = lax.axis_index("s")
  row0 = c * (EHALF // CH) + s * CROWS

  def zrow(i, _):
    for q in range(D // 16):
      rows0[i, pl.ds(q * 16, 16)] = jnp.zeros((16,), jnp.float32)
    return 0

  lax.fori_loop(0, CH, zrow, 0)
  for piece in range(NPT // CH):
    pltpu.sync_copy(rows0, acc.at[pl.ds(s * NPT + piece * CH, CH)])
  plsc.subcore_barrier()

  def scale(rows, j):
    def body(t, _):
      nv = norm2[j, pl.ds(t * 16, 16)]
      for k16 in range(16):
        nk = nv[k16]
        for q in range(D // 16):
          sl = pl.ds(q * 16, 16)
          rows[t * 16 + k16, sl] = rows[t * 16 + k16, sl] * nk
      return 0

    lax.fori_loop(0, CH // 16, body, 0)

  for sb in range(CROWS // SUB):
    rb = row0 + sb * SUB
    pltpu.sync_copy(csrc_hbm.at[pl.ds(rb, SUB)], csrc2)
    pltpu.sync_copy(dst_hbm.at[pl.ds(rb, SUB)], dst2)
    pltpu.sync_copy(norm_hbm.at[pl.ds(rb, SUB)], norm2)
    pltpu.async_copy(xs_hbm.at[csrc2.at[0]], rows0, gsem0)

    def pipe(k, _):
      j0 = 2 * k
      pltpu.make_async_copy(xs_hbm.at[csrc2.at[j0]], rows0, gsem0).wait()
      pltpu.async_copy(xs_hbm.at[csrc2.at[j0 + 1]], rows1, gsem1)
      scale(rows0, j0)
      pltpu.async_copy(rows0, acc.at[dst2.at[j0]], ssem0, add=True)
      pltpu.make_async_copy(xs_hbm.at[csrc2.at[j0 + 1]], rows1, gsem1).wait()
      pltpu.make_async_copy(rows0, acc.at[dst2.at[j0]], ssem0).wait()

      @pl.when(k < SUB // 2 - 1)
      def _():
        pltpu.async_copy(xs_hbm.at[csrc2.at[j0 + 2]], rows0, gsem0)

      scale(rows1, j0 + 1)
      pltpu.async_copy(rows1, acc.at[dst2.at[j0 + 1]], ssem1, add=True)
      pltpu.make_async_copy(rows1, acc.at[dst2.at[j0 + 1]], ssem1).wait()
      return 0

    lax.fori_loop(0, SUB // 2, pipe, 0)
  plsc.subcore_barrier()

  pltpu.sync_copy(acc.at[pl.ds(s * NPT, NPT)],
                  out_hbm.at[pl.ds(c * NPAD + s * NPT, NPT)])


_BN = 1000


def _tc_transform_body(h_ref, w_ref, root_ref, b_ref, xs_ref, base_ref):
  h = h_ref[...]
  base_ref[...] = (
      jnp.dot(h, root_ref[...], preferred_element_type=jnp.float32)
      + b_ref[...]
  )
  for r in range(R):
    for blk in range(NB):
      xs_ref[r, :, blk * BI:(blk + 1) * BI] = jnp.dot(
          h[:, blk * BI:(blk + 1) * BI], w_ref[r, blk],
          preferred_element_type=jnp.float32)


def _tc_transform(h, w, root, b):
  xs, base = pl.pallas_call(
      _tc_transform_body,
      grid=(N // _BN,),
      in_specs=[
          pl.BlockSpec((_BN, D), lambda i: (i, 0)),
          pl.BlockSpec((R, NB, BI, BI), lambda i: (0, 0, 0, 0)),
          pl.BlockSpec((D, D), lambda i: (0, 0)),
          pl.BlockSpec((1, D), lambda i: (0, 0)),
      ],
      out_specs=[
          pl.BlockSpec((R, _BN, D), lambda i: (0, i, 0)),
          pl.BlockSpec((_BN, D), lambda i: (i, 0)),
      ],
      out_shape=[
          jax.ShapeDtypeStruct((R, N, D), jnp.float32),
          jax.ShapeDtypeStruct((N, D), jnp.float32),
      ],
  )(h, w, root, b.reshape(1, D))
  return xs.reshape(R * N, D), base


def _tc_combine(p0, p1, base, relu):
  def body(p0_ref, p1_ref, base_ref, out_ref):
    t = p0_ref[...] + p1_ref[...] + base_ref[...]
    out_ref[...] = jnp.maximum(t, 0.0) if relu else t

  return pl.pallas_call(
      body,
      grid=(N // _BN,),
      in_specs=[pl.BlockSpec((_BN, D), lambda i: (i, 0))] * 3,
      out_specs=pl.BlockSpec((_BN, D), lambda i: (i, 0)),
      out_shape=jax.ShapeDtypeStruct((N, D), jnp.float32),
  )(p0, p1, base)


def kernel(x, edge_index, edge_type, w1, root1, b1, w2, root2, b2):
  src = edge_index[0].astype(jnp.int32)
  dst = edge_index[1].astype(jnp.int32)
  et = edge_type.astype(jnp.int32)
  srcp = jnp.concatenate([src, jnp.zeros((PADE,), jnp.int32)])
  dstp = jnp.concatenate([dst, jnp.zeros((PADE,), jnp.int32)])
  etp = jnp.concatenate([et, jnp.full((PADE,), R, jnp.int32)])

  norm_i, csrc = _sc_prep(srcp, dstp, etp)

  dst2d = dstp.reshape(EROWS, CH)
  csrc2d = csrc.reshape(EROWS, CH)
  norm2d = lax.bitcast_convert_type(norm_i, jnp.float32).reshape(EROWS, CH)

  xs1, base1 = _tc_transform(x, w1, root1, b1)
  parts1 = _sc_message(xs1, csrc2d, dst2d, norm2d)
  h = _tc_combine(parts1[:N], parts1[NPAD:NPAD + N], base1, relu=True)

  xs2, base2 = _tc_transform(h, w2, root2, b2)
  parts2 = _sc_message(xs2, csrc2d, dst2d, norm2d)
  return _tc_combine(parts2[:N], parts2[NPAD:NPAD + N], base2, relu=False)
